# baseline (speedup 1.0000x reference)
import functools

import jax
import jax.numpy as jnp
from jax import lax
from jax.experimental import pallas as pl
from jax.experimental.pallas import tpu as pltpu
from jax.experimental.pallas import tpu_sc as plsc

N = 100000
D = 128
S = 512
NC = 2
NSUB = 16
NW = NC * NSUB
BLK = 128
GPB = BLK // 16
TPB = 13
SC_ROWS = NW * TPB * BLK
TCB = 1024
TCW = 128
REMT = 672
TCG = (N - SC_ROWS - REMT) // TCB
TC_ROWS = TCG * TCB
SPAD = S + TCW
REM = N - SC_ROWS - TC_ROWS
NREM_TILES = REM // 32
REM0 = SC_ROWS + TC_ROWS
NCH = D // 16
BSTAGE = TPB * BLK
BBUF = BSTAGE + 32 + 16
MAXI = (TPB + 2) // 3


def _seg_body(x_hbm, b_hbm, sums_out, cnt_out,
              xbuf0, xbuf1, xbuf2, bbuf, acc, cnt, sem0, sem1, sem2):
    cid = lax.axis_index("c")
    sid = lax.axis_index("s")
    wid = sid * NC + cid
    rowstart = pl.multiple_of(wid * TPB * BLK, BLK)

    bufs = [(xbuf0, sem0), (xbuf1, sem1), (xbuf2, sem2)]

    zero = jnp.zeros((16,), jnp.float32)
    one0 = (lax.iota(jnp.int32, 16) == 0).astype(jnp.float32)
    lanes = lax.iota(jnp.int32, 16)

    def issue(blk, buf, sem):
        roff = pl.multiple_of(rowstart + blk * BLK, BLK)
        pltpu.async_copy(x_hbm.at[pl.ds(roff, BLK), :], buf, sem)

    def wait(buf, sem):
        pltpu.make_async_copy(x_hbm.at[pl.ds(0, BLK), :], buf, sem).wait()

    issue(0, xbuf0, sem0)
    issue(1, xbuf1, sem1)

    def zero_body(r, carry):
        for f in range(NCH):
            acc[r, pl.ds(f * 16, 16)] = zero
        return carry

    lax.fori_loop(0, S, zero_body, None)
    for j in range((S + 16) // 16):
        cnt[pl.ds(j * 16, 16)] = zero

    pltpu.sync_copy(b_hbm.at[pl.ds(rowstart, BSTAGE)], bbuf.at[pl.ds(0, BSTAGE)])

    def flush(A, cnt_run, cur_seg):
        ssafe = jnp.maximum(cur_seg, 0)
        for f in range(NCH):
            plsc.addupdate(acc.at[ssafe, pl.ds(f * 16, 16)], A[f])
        plsc.addupdate(cnt.at[pl.ds(ssafe, 16)], one0 * cnt_run)

    def group_step(xref, xrow0, bbase, C):
        A, cnt_run, cur_seg = C
        b = bbuf[pl.ds(bbase, 16)]
        ndiff = jnp.sum((b != cur_seg).astype(jnp.int32))

        def fast(C):
            A, cnt_run, cur_seg = C
            newA = []
            for f in range(NCH):
                loads = [xref[xrow0 + i, pl.ds(f * 16, 16)] for i in range(16)]
                while len(loads) > 1:
                    loads = [loads[i] + loads[i + 1]
                             for i in range(0, len(loads) - 1, 2)] + (
                                 [loads[-1]] if len(loads) % 2 else [])
                newA.append(A[f] + loads[0])
            return (tuple(newA), cnt_run + 16.0, cur_seg)

        def slow(C):
            def scond(st):
                return st[0] < 16

            def sbody(st):
                r, A, cnt_run, cur_seg = st
                neq = (b != cur_seg) & (lanes >= r)
                after = plsc.cummax(neq.astype(jnp.int32))
                p = 16 - jnp.sum(after)

                def rbody(rr, A):
                    return tuple(A[f] + xref[xrow0 + rr, pl.ds(f * 16, 16)]
                                 for f in range(NCH))

                A = lax.fori_loop(r, p, rbody, A)
                cnt_run = cnt_run + (p - r).astype(jnp.float32)

                def switch(args):
                    A, cnt_run, cur_seg = args
                    flush(A, cnt_run, cur_seg)
                    new_seg = jnp.sum(jnp.where(lanes == p, b, 0))
                    return (tuple(zero for _ in range(NCH)), 0.0, new_seg)

                A, cnt_run, cur_seg = lax.cond(
                    p < 16, switch, lambda a: a, (A, cnt_run, cur_seg))
                return (p, A, cnt_run, cur_seg)

            r0 = jnp.int32(0)
            _, A, cnt_run, cur_seg = lax.while_loop(scond, sbody,
                                                    (r0, *C))
            return (A, cnt_run, cur_seg)

        return lax.cond(ndiff == 0, fast, slow, C)

    def process_block(xref, blk, C):
        def gbody(g, C):
            return group_step(xref, g * 16, blk * BLK + g * 16, C)

        return lax.fori_loop(0, GPB, gbody, C)

    def guarded(pred, fn, C):
        return lax.cond(pred, fn, lambda c: c, C)

    A0 = tuple(zero for _ in range(NCH))
    C = (A0, 0.0, jnp.int32(-1))

    def step3(i, C):
        for j in range(3):
            blk = i * 3 + j
            buf, sem = bufs[j]
            nbuf, nsem = bufs[(j + 2) % 3]

            @pl.when(blk < TPB)
            def _():
                wait(buf, sem)

            @pl.when(blk + 2 < TPB)
            def _():
                issue(blk + 2, nbuf, nsem)

            C = guarded(blk < TPB,
                        functools.partial(process_block, buf, blk), C)
        return C

    C = lax.fori_loop(0, MAXI, step3, C)

    def rem_fn(C):
        tstart = pl.multiple_of(REM0 + wid * 32, 32)
        pltpu.sync_copy(x_hbm.at[pl.ds(tstart, 32), :],
                        xbuf0.at[pl.ds(0, 32), :])
        pltpu.sync_copy(b_hbm.at[pl.ds(tstart, 32)],
                        bbuf.at[pl.ds(BSTAGE, 32)])
        for g in range(2):
            C = group_step(xbuf0, g * 16, BSTAGE + g * 16, C)
        return C

    C = guarded(wid < NREM_TILES, rem_fn, C)

    A, cnt_run, cur_seg = C
    flush(A, cnt_run, cur_seg)

    pltpu.sync_copy(acc, sums_out.at[wid])
    pltpu.sync_copy(cnt, cnt_out.at[wid])


_seg_kernel = functools.partial(
    pl.kernel,
    mesh=plsc.VectorSubcoreMesh(core_axis_name="c", subcore_axis_name="s"),
    compiler_params=pltpu.CompilerParams(needs_layout_passes=False),
    out_type=[
        jax.ShapeDtypeStruct((NW, S, D), jnp.float32),
        jax.ShapeDtypeStruct((NW, S + 16), jnp.float32),
    ],
    scratch_types=[
        pltpu.VMEM((BLK, D), jnp.float32),
        pltpu.VMEM((BLK, D), jnp.float32),
        pltpu.VMEM((BLK, D), jnp.float32),
        pltpu.VMEM((BBUF,), jnp.int32),
        pltpu.VMEM((S, D), jnp.float32),
        pltpu.VMEM((S + 16,), jnp.float32),
        pltpu.SemaphoreType.DMA,
        pltpu.SemaphoreType.DMA,
        pltpu.SemaphoreType.DMA,
    ],
)(_seg_body)


def _tc_body(ids_ref, x_ref, sums_ref, cnt_ref):
    c = pl.program_id(0)

    @pl.when(c == 0)
    def _():
        sums_ref[...] = jnp.zeros_like(sums_ref)
        cnt_ref[...] = jnp.zeros_like(cnt_ref)

    ids = ids_ref[...]
    lo_id = jnp.min(ids)
    hi_id = jnp.max(ids)
    base = pl.multiple_of((lo_id // 8) * 8, 8)
    xb = x_ref[...].astype(jnp.bfloat16)
    for k in range(S // TCW):
        wbase = pl.multiple_of(base + k * TCW, 8)

        @pl.when(wbase <= hi_id)
        def _():
            segs = wbase + lax.broadcasted_iota(jnp.int32, (TCW, TCB), 0)
            hit = segs == ids
            oh = hit.astype(jnp.bfloat16)
            ps = jnp.dot(oh, xb, preferred_element_type=jnp.float32)
            sums_ref[pl.ds(wbase, TCW), :] += ps
            cnt_ref[pl.ds(wbase, TCW), :] += jnp.sum(
                hit.astype(jnp.float32), axis=1, keepdims=True)


def _combine_body(sums_ref, cnt_ref, tcs_ref, tcc_ref, out_ref):
    s = jnp.sum(sums_ref[...], axis=0) + tcs_ref[:S, :]
    c = (jnp.sum(cnt_ref[...], axis=0)[:S, None]
         + tcc_ref[:S, :])
    out_ref[...] = s / jnp.clip(c, 1.0, None)


def kernel(x, batch):
    sums, cnts = _seg_kernel(x, batch)
    tc_sums, tc_cnt = pl.pallas_call(
        _tc_body,
        grid=(TCG,),
        in_specs=[
            pl.BlockSpec((1, TCB), lambda c: (0, c + SC_ROWS // TCB)),
            pl.BlockSpec((TCB, D), lambda c: (c + SC_ROWS // TCB, 0)),
        ],
        out_specs=[
            pl.BlockSpec((SPAD, D), lambda c: (0, 0)),
            pl.BlockSpec((SPAD, 1), lambda c: (0, 0)),
        ],
        out_shape=[
            jax.ShapeDtypeStruct((SPAD, D), jnp.float32),
            jax.ShapeDtypeStruct((SPAD, 1), jnp.float32),
        ],
    )(batch.reshape(1, N), x)
    return pl.pallas_call(
        _combine_body,
        out_shape=jax.ShapeDtypeStruct((S, D), jnp.float32),
    )(sums, cnts, tc_sums, tc_cnt)


# speedup vs baseline: 1.1702x; 1.1702x over previous
import functools

import jax
import jax.numpy as jnp
from jax import lax
from jax.experimental import pallas as pl
from jax.experimental.pallas import tpu as pltpu
from jax.experimental.pallas import tpu_sc as plsc

N = 100000
D = 128
S = 512
NC = 2
NSUB = 16
NW = NC * NSUB
BLK = 128
GPB = BLK // 16
TPB = 16
SC_ROWS = NW * TPB * BLK
TCB = 1024
TCW = 128
REMT = 672
TCG = (N - SC_ROWS - REMT) // TCB
TC_ROWS = TCG * TCB
SPAD = S + TCW
REM = N - SC_ROWS - TC_ROWS
NREM_TILES = REM // 32
REM0 = SC_ROWS + TC_ROWS
NCH = D // 16
BSTAGE = TPB * BLK
BBUF = BSTAGE + 32 + 16
MAXI = (TPB + 2) // 3


def _seg_body(x_hbm, b_hbm, sums_out, cnt_out,
              xbuf0, xbuf1, xbuf2, bbuf, acc, cnt, sem0, sem1, sem2):
    cid = lax.axis_index("c")
    sid = lax.axis_index("s")
    wid = sid * NC + cid
    rowstart = pl.multiple_of(wid * TPB * BLK, BLK)

    bufs = [(xbuf0, sem0), (xbuf1, sem1), (xbuf2, sem2)]

    zero = jnp.zeros((16,), jnp.float32)
    one0 = (lax.iota(jnp.int32, 16) == 0).astype(jnp.float32)
    lanes = lax.iota(jnp.int32, 16)

    def issue(blk, buf, sem):
        roff = pl.multiple_of(rowstart + blk * BLK, BLK)
        pltpu.async_copy(x_hbm.at[pl.ds(roff, BLK), :], buf, sem)

    def wait(buf, sem):
        pltpu.make_async_copy(x_hbm.at[pl.ds(0, BLK), :], buf, sem).wait()

    issue(0, xbuf0, sem0)
    issue(1, xbuf1, sem1)

    def zero_body(r, carry):
        for f in range(NCH):
            acc[r, pl.ds(f * 16, 16)] = zero
        return carry

    lax.fori_loop(0, S, zero_body, None)
    for j in range((S + 16) // 16):
        cnt[pl.ds(j * 16, 16)] = zero

    pltpu.sync_copy(b_hbm.at[pl.ds(rowstart, BSTAGE)], bbuf.at[pl.ds(0, BSTAGE)])

    def flush(A, cnt_run, cur_seg):
        ssafe = jnp.maximum(cur_seg, 0)
        for f in range(NCH):
            plsc.addupdate(acc.at[ssafe, pl.ds(f * 16, 16)], A[f])
        plsc.addupdate(cnt.at[pl.ds(ssafe, 16)], one0 * cnt_run)

    def group_step(xref, xrow0, bbase, C):
        A, cnt_run, cur_seg = C
        b = bbuf[pl.ds(bbase, 16)]
        ndiff = jnp.sum((b != cur_seg).astype(jnp.int32))

        def fast(C):
            A, cnt_run, cur_seg = C
            newA = []
            for f in range(NCH):
                loads = [xref[xrow0 + i, pl.ds(f * 16, 16)] for i in range(16)]
                while len(loads) > 1:
                    loads = [loads[i] + loads[i + 1]
                             for i in range(0, len(loads) - 1, 2)] + (
                                 [loads[-1]] if len(loads) % 2 else [])
                newA.append(A[f] + loads[0])
            return (tuple(newA), cnt_run + 16.0, cur_seg)

        def slow(C):
            def scond(st):
                return st[0] < 16

            def sbody(st):
                r, A, cnt_run, cur_seg = st
                neq = (b != cur_seg) & (lanes >= r)
                after = plsc.cummax(neq.astype(jnp.int32))
                p = 16 - jnp.sum(after)

                def rbody(rr, A):
                    return tuple(A[f] + xref[xrow0 + rr, pl.ds(f * 16, 16)]
                                 for f in range(NCH))

                A = lax.fori_loop(r, p, rbody, A)
                cnt_run = cnt_run + (p - r).astype(jnp.float32)

                def switch(args):
                    A, cnt_run, cur_seg = args
                    flush(A, cnt_run, cur_seg)
                    new_seg = jnp.sum(jnp.where(lanes == p, b, 0))
                    return (tuple(zero for _ in range(NCH)), 0.0, new_seg)

                A, cnt_run, cur_seg = lax.cond(
                    p < 16, switch, lambda a: a, (A, cnt_run, cur_seg))
                return (p, A, cnt_run, cur_seg)

            r0 = jnp.int32(0)
            _, A, cnt_run, cur_seg = lax.while_loop(scond, sbody,
                                                    (r0, *C))
            return (A, cnt_run, cur_seg)

        return lax.cond(ndiff == 0, fast, slow, C)

    def process_block(xref, blk, C):
        def gbody(g, C):
            return group_step(xref, g * 16, blk * BLK + g * 16, C)

        return lax.fori_loop(0, GPB, gbody, C)

    def guarded(pred, fn, C):
        return lax.cond(pred, fn, lambda c: c, C)

    A0 = tuple(zero for _ in range(NCH))
    C = (A0, 0.0, jnp.int32(-1))

    def step3(i, C):
        for j in range(3):
            blk = i * 3 + j
            buf, sem = bufs[j]
            nbuf, nsem = bufs[(j + 2) % 3]

            @pl.when(blk < TPB)
            def _():
                wait(buf, sem)

            @pl.when(blk + 2 < TPB)
            def _():
                issue(blk + 2, nbuf, nsem)

            C = guarded(blk < TPB,
                        functools.partial(process_block, buf, blk), C)
        return C

    C = lax.fori_loop(0, MAXI, step3, C)

    def rem_fn(C):
        tstart = pl.multiple_of(REM0 + wid * 32, 32)
        pltpu.sync_copy(x_hbm.at[pl.ds(tstart, 32), :],
                        xbuf0.at[pl.ds(0, 32), :])
        pltpu.sync_copy(b_hbm.at[pl.ds(tstart, 32)],
                        bbuf.at[pl.ds(BSTAGE, 32)])
        for g in range(2):
            C = group_step(xbuf0, g * 16, BSTAGE + g * 16, C)
        return C

    C = guarded(wid < NREM_TILES, rem_fn, C)

    A, cnt_run, cur_seg = C
    flush(A, cnt_run, cur_seg)

    pltpu.sync_copy(acc, sums_out.at[wid])
    pltpu.sync_copy(cnt, cnt_out.at[wid])


_seg_kernel = functools.partial(
    pl.kernel,
    mesh=plsc.VectorSubcoreMesh(core_axis_name="c", subcore_axis_name="s"),
    compiler_params=pltpu.CompilerParams(needs_layout_passes=False),
    out_type=[
        jax.ShapeDtypeStruct((NW, S, D), jnp.float32),
        jax.ShapeDtypeStruct((NW, S + 16), jnp.float32),
    ],
    scratch_types=[
        pltpu.VMEM((BLK, D), jnp.float32),
        pltpu.VMEM((BLK, D), jnp.float32),
        pltpu.VMEM((BLK, D), jnp.float32),
        pltpu.VMEM((BBUF,), jnp.int32),
        pltpu.VMEM((S, D), jnp.float32),
        pltpu.VMEM((S + 16,), jnp.float32),
        pltpu.SemaphoreType.DMA,
        pltpu.SemaphoreType.DMA,
        pltpu.SemaphoreType.DMA,
    ],
)(_seg_body)


def _tc_body(ids_ref, x_ref, sums_ref, cnt_ref):
    c = pl.program_id(0)

    @pl.when(c == 0)
    def _():
        sums_ref[...] = jnp.zeros_like(sums_ref)
        cnt_ref[...] = jnp.zeros_like(cnt_ref)

    ids = ids_ref[...]
    lo_id = jnp.min(ids)
    hi_id = jnp.max(ids)
    base = pl.multiple_of((lo_id // 8) * 8, 8)
    xb = x_ref[...].astype(jnp.bfloat16)
    for k in range(S // TCW):
        wbase = pl.multiple_of(base + k * TCW, 8)

        @pl.when(wbase <= hi_id)
        def _():
            segs = wbase + lax.broadcasted_iota(jnp.int32, (TCW, TCB), 0)
            hit = segs == ids
            oh = hit.astype(jnp.bfloat16)
            ps = jnp.dot(oh, xb, preferred_element_type=jnp.float32)
            sums_ref[pl.ds(wbase, TCW), :] += ps
            cnt_ref[pl.ds(wbase, TCW), :] += jnp.sum(
                hit.astype(jnp.float32), axis=1, keepdims=True)


def _combine_body(sums_ref, cnt_ref, tcs_ref, tcc_ref, out_ref):
    s = jnp.sum(sums_ref[...], axis=0) + tcs_ref[:S, :]
    c = (jnp.sum(cnt_ref[...], axis=0)[:S, None]
         + tcc_ref[:S, :])
    out_ref[...] = s / jnp.clip(c, 1.0, None)


def kernel(x, batch):
    sums, cnts = _seg_kernel(x, batch)
    tc_sums, tc_cnt = pl.pallas_call(
        _tc_body,
        grid=(TCG,),
        in_specs=[
            pl.BlockSpec((1, TCB), lambda c: (0, c + SC_ROWS // TCB)),
            pl.BlockSpec((TCB, D), lambda c: (c + SC_ROWS // TCB, 0)),
        ],
        out_specs=[
            pl.BlockSpec((SPAD, D), lambda c: (0, 0)),
            pl.BlockSpec((SPAD, 1), lambda c: (0, 0)),
        ],
        out_shape=[
            jax.ShapeDtypeStruct((SPAD, D), jnp.float32),
            jax.ShapeDtypeStruct((SPAD, 1), jnp.float32),
        ],
    )(batch.reshape(1, N), x)
    return pl.pallas_call(
        _combine_body,
        out_shape=jax.ShapeDtypeStruct((S, D), jnp.float32),
    )(sums, cnts, tc_sums, tc_cnt)


# speedup vs baseline: 1.1858x; 1.0133x over previous
import functools

import jax
import jax.numpy as jnp
from jax import lax
from jax.experimental import pallas as pl
from jax.experimental.pallas import tpu as pltpu
from jax.experimental.pallas import tpu_sc as plsc

N = 100000
D = 128
S = 512
NC = 2
NSUB = 16
NW = NC * NSUB
BLK = 128
GPB = BLK // 16
TPB = 16
SC_ROWS = NW * TPB * BLK
TCB = 1024
TCW = 64
REMT = 672
TCG = (N - SC_ROWS - REMT) // TCB
TC_ROWS = TCG * TCB
SPAD = S + TCW
REM = N - SC_ROWS - TC_ROWS
NREM_TILES = REM // 32
REM0 = SC_ROWS + TC_ROWS
NCH = D // 16
BSTAGE = TPB * BLK
BBUF = BSTAGE + 32 + 16
MAXI = (TPB + 2) // 3


def _seg_body(x_hbm, b_hbm, sums_out, cnt_out,
              xbuf0, xbuf1, xbuf2, bbuf, acc, cnt, sem0, sem1, sem2):
    cid = lax.axis_index("c")
    sid = lax.axis_index("s")
    wid = sid * NC + cid
    rowstart = pl.multiple_of(wid * TPB * BLK, BLK)

    bufs = [(xbuf0, sem0), (xbuf1, sem1), (xbuf2, sem2)]

    zero = jnp.zeros((16,), jnp.float32)
    one0 = (lax.iota(jnp.int32, 16) == 0).astype(jnp.float32)
    lanes = lax.iota(jnp.int32, 16)

    def issue(blk, buf, sem):
        roff = pl.multiple_of(rowstart + blk * BLK, BLK)
        pltpu.async_copy(x_hbm.at[pl.ds(roff, BLK), :], buf, sem)

    def wait(buf, sem):
        pltpu.make_async_copy(x_hbm.at[pl.ds(0, BLK), :], buf, sem).wait()

    issue(0, xbuf0, sem0)
    issue(1, xbuf1, sem1)

    def zero_body(r, carry):
        for f in range(NCH):
            acc[r, pl.ds(f * 16, 16)] = zero
        return carry

    lax.fori_loop(0, S, zero_body, None)
    for j in range((S + 16) // 16):
        cnt[pl.ds(j * 16, 16)] = zero

    pltpu.sync_copy(b_hbm.at[pl.ds(rowstart, BSTAGE)], bbuf.at[pl.ds(0, BSTAGE)])

    def flush(A, cnt_run, cur_seg):
        ssafe = jnp.maximum(cur_seg, 0)
        for f in range(NCH):
            plsc.addupdate(acc.at[ssafe, pl.ds(f * 16, 16)], A[f])
        plsc.addupdate(cnt.at[pl.ds(ssafe, 16)], one0 * cnt_run)

    def group_step(xref, xrow0, bbase, C):
        A, cnt_run, cur_seg = C
        b = bbuf[pl.ds(bbase, 16)]
        ndiff = jnp.sum((b != cur_seg).astype(jnp.int32))

        def fast(C):
            A, cnt_run, cur_seg = C
            newA = []
            for f in range(NCH):
                loads = [xref[xrow0 + i, pl.ds(f * 16, 16)] for i in range(16)]
                while len(loads) > 1:
                    loads = [loads[i] + loads[i + 1]
                             for i in range(0, len(loads) - 1, 2)] + (
                                 [loads[-1]] if len(loads) % 2 else [])
                newA.append(A[f] + loads[0])
            return (tuple(newA), cnt_run + 16.0, cur_seg)

        def slow(C):
            def scond(st):
                return st[0] < 16

            def sbody(st):
                r, A, cnt_run, cur_seg = st
                neq = (b != cur_seg) & (lanes >= r)
                after = plsc.cummax(neq.astype(jnp.int32))
                p = 16 - jnp.sum(after)

                def rbody(rr, A):
                    return tuple(A[f] + xref[xrow0 + rr, pl.ds(f * 16, 16)]
                                 for f in range(NCH))

                A = lax.fori_loop(r, p, rbody, A)
                cnt_run = cnt_run + (p - r).astype(jnp.float32)

                def switch(args):
                    A, cnt_run, cur_seg = args
                    flush(A, cnt_run, cur_seg)
                    new_seg = jnp.sum(jnp.where(lanes == p, b, 0))
                    return (tuple(zero for _ in range(NCH)), 0.0, new_seg)

                A, cnt_run, cur_seg = lax.cond(
                    p < 16, switch, lambda a: a, (A, cnt_run, cur_seg))
                return (p, A, cnt_run, cur_seg)

            r0 = jnp.int32(0)
            _, A, cnt_run, cur_seg = lax.while_loop(scond, sbody,
                                                    (r0, *C))
            return (A, cnt_run, cur_seg)

        return lax.cond(ndiff == 0, fast, slow, C)

    def process_block(xref, blk, C):
        def gbody(g, C):
            return group_step(xref, g * 16, blk * BLK + g * 16, C)

        return lax.fori_loop(0, GPB, gbody, C)

    def guarded(pred, fn, C):
        return lax.cond(pred, fn, lambda c: c, C)

    A0 = tuple(zero for _ in range(NCH))
    C = (A0, 0.0, jnp.int32(-1))

    def step3(i, C):
        for j in range(3):
            blk = i * 3 + j
            buf, sem = bufs[j]
            nbuf, nsem = bufs[(j + 2) % 3]

            @pl.when(blk < TPB)
            def _():
                wait(buf, sem)

            @pl.when(blk + 2 < TPB)
            def _():
                issue(blk + 2, nbuf, nsem)

            C = guarded(blk < TPB,
                        functools.partial(process_block, buf, blk), C)
        return C

    C = lax.fori_loop(0, MAXI, step3, C)

    def rem_fn(C):
        tstart = pl.multiple_of(REM0 + wid * 32, 32)
        pltpu.sync_copy(x_hbm.at[pl.ds(tstart, 32), :],
                        xbuf0.at[pl.ds(0, 32), :])
        pltpu.sync_copy(b_hbm.at[pl.ds(tstart, 32)],
                        bbuf.at[pl.ds(BSTAGE, 32)])
        for g in range(2):
            C = group_step(xbuf0, g * 16, BSTAGE + g * 16, C)
        return C

    C = guarded(wid < NREM_TILES, rem_fn, C)

    A, cnt_run, cur_seg = C
    flush(A, cnt_run, cur_seg)

    pltpu.sync_copy(acc, sums_out.at[wid])
    pltpu.sync_copy(cnt, cnt_out.at[wid])


_seg_kernel = functools.partial(
    pl.kernel,
    mesh=plsc.VectorSubcoreMesh(core_axis_name="c", subcore_axis_name="s"),
    compiler_params=pltpu.CompilerParams(needs_layout_passes=False),
    out_type=[
        jax.ShapeDtypeStruct((NW, S, D), jnp.float32),
        jax.ShapeDtypeStruct((NW, S + 16), jnp.float32),
    ],
    scratch_types=[
        pltpu.VMEM((BLK, D), jnp.float32),
        pltpu.VMEM((BLK, D), jnp.float32),
        pltpu.VMEM((BLK, D), jnp.float32),
        pltpu.VMEM((BBUF,), jnp.int32),
        pltpu.VMEM((S, D), jnp.float32),
        pltpu.VMEM((S + 16,), jnp.float32),
        pltpu.SemaphoreType.DMA,
        pltpu.SemaphoreType.DMA,
        pltpu.SemaphoreType.DMA,
    ],
)(_seg_body)


def _tc_body(ids_ref, x_ref, sums_ref, cnt_ref):
    c = pl.program_id(0)

    @pl.when(c == 0)
    def _():
        sums_ref[...] = jnp.zeros_like(sums_ref)
        cnt_ref[...] = jnp.zeros_like(cnt_ref)

    ids = ids_ref[...]
    lo_id = jnp.min(ids)
    hi_id = jnp.max(ids)
    base = pl.multiple_of((lo_id // 8) * 8, 8)
    xb = x_ref[...].astype(jnp.bfloat16)
    for k in range(S // TCW):
        wbase = pl.multiple_of(base + k * TCW, 8)

        @pl.when(wbase <= hi_id)
        def _():
            segs = wbase + lax.broadcasted_iota(jnp.int32, (TCW, TCB), 0)
            hit = segs == ids
            oh = hit.astype(jnp.bfloat16)
            ps = jnp.dot(oh, xb, preferred_element_type=jnp.float32)
            sums_ref[pl.ds(wbase, TCW), :] += ps
            cnt_ref[pl.ds(wbase, TCW), :] += jnp.sum(
                hit.astype(jnp.float32), axis=1, keepdims=True)


def _combine_body(sums_ref, cnt_ref, tcs_ref, tcc_ref, out_ref):
    s = jnp.sum(sums_ref[...], axis=0) + tcs_ref[:S, :]
    c = (jnp.sum(cnt_ref[...], axis=0)[:S, None]
         + tcc_ref[:S, :])
    out_ref[...] = s / jnp.clip(c, 1.0, None)


def kernel(x, batch):
    sums, cnts = _seg_kernel(x, batch)
    tc_sums, tc_cnt = pl.pallas_call(
        _tc_body,
        grid=(TCG,),
        in_specs=[
            pl.BlockSpec((1, TCB), lambda c: (0, c + SC_ROWS // TCB)),
            pl.BlockSpec((TCB, D), lambda c: (c + SC_ROWS // TCB, 0)),
        ],
        out_specs=[
            pl.BlockSpec((SPAD, D), lambda c: (0, 0)),
            pl.BlockSpec((SPAD, 1), lambda c: (0, 0)),
        ],
        out_shape=[
            jax.ShapeDtypeStruct((SPAD, D), jnp.float32),
            jax.ShapeDtypeStruct((SPAD, 1), jnp.float32),
        ],
    )(batch.reshape(1, N), x)
    return pl.pallas_call(
        _combine_body,
        out_shape=jax.ShapeDtypeStruct((S, D), jnp.float32),
    )(sums, cnts, tc_sums, tc_cnt)
